# initial kernel scaffold (unmeasured)
import jax
import jax.numpy as jnp
from jax import lax
from jax.experimental import pallas as pl
from jax.experimental.pallas import tpu as pltpu

N_DEV = 4


def _ag_body(x_ref, out_ref, local_sem, send_sems, recv_sems):
    my = lax.axis_index("i")
    left = (my - 1) % N_DEV
    right = (my + 1) % N_DEV
    m_per = x_ref.shape[0]

    barrier_sem = pltpu.get_barrier_semaphore()
    for nbr in [left, right]:
        pl.semaphore_signal(
            barrier_sem, inc=1,
            device_id=(nbr,), device_id_type=pl.DeviceIdType.MESH,
        )
    pl.semaphore_wait(barrier_sem, 2)

    cp = pltpu.make_async_copy(
        x_ref, out_ref.at[pl.ds(my * m_per, m_per), :], local_sem
    )
    cp.start()
    cp.wait()

    for h in range(N_DEV - 1):
        send_c = (my - h) % N_DEV
        recv_c = (my - h - 1) % N_DEV
        send = pltpu.make_async_remote_copy(
            src_ref=out_ref.at[pl.ds(send_c * m_per, m_per), :],
            dst_ref=out_ref.at[pl.ds(send_c * m_per, m_per), :],
            send_sem=send_sems.at[h],
            recv_sem=recv_sems.at[h],
            device_id=(right,),
            device_id_type=pl.DeviceIdType.MESH,
        )
        send.start()
        recv = pltpu.make_async_remote_copy(
            src_ref=out_ref.at[pl.ds(recv_c * m_per, m_per), :],
            dst_ref=out_ref.at[pl.ds(recv_c * m_per, m_per), :],
            send_sem=send_sems.at[h],
            recv_sem=recv_sems.at[h],
            device_id=(left,),
            device_id_type=pl.DeviceIdType.MESH,
        )
        recv.wait_recv()
        send.wait_send()


def _all_gather_x(x_shard):
    m_per, k = x_shard.shape
    return pl.pallas_call(
        _ag_body,
        out_shape=jax.ShapeDtypeStruct((N_DEV * m_per, k), x_shard.dtype),
        in_specs=[pl.BlockSpec(memory_space=pltpu.ANY)],
        out_specs=pl.BlockSpec(memory_space=pltpu.ANY),
        scratch_shapes=[
            pltpu.SemaphoreType.DMA,
            pltpu.SemaphoreType.DMA((N_DEV - 1,)),
            pltpu.SemaphoreType.DMA((N_DEV - 1,)),
        ],
        compiler_params=pltpu.CompilerParams(collective_id=0),
    )(x_shard)


def _gelu(y):
    c = 0.7978845608028654
    return 0.5 * y * (1.0 + jnp.tanh(c * (y + 0.044715 * y * y * y)))


def kernel(x, w_mat):
    full_x = _all_gather_x(x)
    y = jnp.dot(full_x, w_mat, preferred_element_type=jnp.float32)
    return _gelu(y).astype(jnp.float32)


# baseline (device time: 4407632 ns/iter reference)
import jax
import jax.numpy as jnp
from jax import lax
from jax.experimental import pallas as pl
from jax.experimental.pallas import tpu as pltpu

N_DEV = 4


def _ag_body(x_ref, out_ref, local_sem, send_sems, recv_sems):
    my = lax.axis_index("i")
    left = (my - 1) % N_DEV
    right = (my + 1) % N_DEV
    m_per = x_ref.shape[0]

    barrier_sem = pltpu.get_barrier_semaphore()
    for nbr in [left, right]:
        pl.semaphore_signal(
            barrier_sem, inc=1,
            device_id=(nbr,), device_id_type=pl.DeviceIdType.MESH,
        )
    pl.semaphore_wait(barrier_sem, 2)

    cp = pltpu.make_async_copy(
        x_ref, out_ref.at[pl.ds(my * m_per, m_per), :], local_sem
    )
    cp.start()
    cp.wait()

    for h in range(N_DEV - 1):
        send_c = (my - h) % N_DEV
        recv_c = (my - h - 1) % N_DEV
        send = pltpu.make_async_remote_copy(
            src_ref=out_ref.at[pl.ds(send_c * m_per, m_per), :],
            dst_ref=out_ref.at[pl.ds(send_c * m_per, m_per), :],
            send_sem=send_sems.at[h],
            recv_sem=recv_sems.at[h],
            device_id=(right,),
            device_id_type=pl.DeviceIdType.MESH,
        )
        send.start()
        recv = pltpu.make_async_remote_copy(
            src_ref=out_ref.at[pl.ds(recv_c * m_per, m_per), :],
            dst_ref=out_ref.at[pl.ds(recv_c * m_per, m_per), :],
            send_sem=send_sems.at[h],
            recv_sem=recv_sems.at[h],
            device_id=(left,),
            device_id_type=pl.DeviceIdType.MESH,
        )
        recv.wait_recv()
        send.wait_send()


def _all_gather_x(x_shard):
    m_per, k = x_shard.shape
    return pl.pallas_call(
        _ag_body,
        out_shape=jax.ShapeDtypeStruct((N_DEV * m_per, k), x_shard.dtype),
        in_specs=[pl.BlockSpec(memory_space=pl.ANY)],
        out_specs=pl.BlockSpec(memory_space=pl.ANY),
        scratch_shapes=[
            pltpu.SemaphoreType.DMA,
            pltpu.SemaphoreType.DMA((N_DEV - 1,)),
            pltpu.SemaphoreType.DMA((N_DEV - 1,)),
        ],
        compiler_params=pltpu.CompilerParams(collective_id=0),
    )(x_shard)


def _gelu(y):
    c = 0.7978845608028654
    return 0.5 * y * (1.0 + jnp.tanh(c * (y + 0.044715 * y * y * y)))


def kernel(x, w_mat):
    full_x = _all_gather_x(x)
    y = jnp.dot(full_x, w_mat, preferred_element_type=jnp.float32)
    return _gelu(y).astype(jnp.float32)


# device time: 1515720 ns/iter; 2.9079x vs baseline; 2.9079x over previous
import jax
import jax.numpy as jnp
from jax import lax
from jax.experimental import pallas as pl
from jax.experimental.pallas import tpu as pltpu

N_DEV = 4


def _neighbor_barrier(left, right):
    barrier_sem = pltpu.get_barrier_semaphore()
    for nbr in [left, right]:
        pl.semaphore_signal(
            barrier_sem, inc=1,
            device_id=(nbr,), device_id_type=pl.DeviceIdType.MESH,
        )
    pl.semaphore_wait(barrier_sem, 2)


def _rdma(src, dst, send_sem, recv_sem, dev):
    return pltpu.make_async_remote_copy(
        src_ref=src, dst_ref=dst, send_sem=send_sem, recv_sem=recv_sem,
        device_id=(dev,), device_id_type=pl.DeviceIdType.MESH,
    )


def _ag_w_body(w_ref, wg_ref, local_sem, send_sems, recv_sems):
    my = lax.axis_index("i")
    left = (my - 1) % N_DEV
    right = (my + 1) % N_DEV
    k, n_per = w_ref.shape
    half = k // 2

    def col(c):
        return pl.ds(c * n_per, n_per)

    _neighbor_barrier(left, right)

    cp = pltpu.make_async_copy(w_ref, wg_ref.at[:, col(my)], local_sem)
    cp.start()

    s_r1 = _rdma(w_ref, wg_ref.at[:, col(my)],
                 send_sems.at[0], recv_sems.at[0], right)
    s_l1 = _rdma(w_ref, wg_ref.at[:, col(my)],
                 send_sems.at[1], recv_sems.at[1], left)
    s_r1.start()
    s_l1.start()

    r_l1 = _rdma(wg_ref.at[:, col(left)], wg_ref.at[:, col(left)],
                 send_sems.at[0], recv_sems.at[0], left)
    r_l1.wait_recv()
    s_r2 = _rdma(wg_ref.at[pl.ds(0, half), col(left)],
                 wg_ref.at[pl.ds(0, half), col(left)],
                 send_sems.at[2], recv_sems.at[2], right)
    s_r2.start()

    r_r1 = _rdma(wg_ref.at[:, col(right)], wg_ref.at[:, col(right)],
                 send_sems.at[1], recv_sems.at[1], right)
    r_r1.wait_recv()
    s_l2 = _rdma(wg_ref.at[pl.ds(half, half), col(right)],
                 wg_ref.at[pl.ds(half, half), col(right)],
                 send_sems.at[3], recv_sems.at[3], left)
    s_l2.start()

    opp = (my + 2) % N_DEV
    r_l2 = _rdma(wg_ref.at[pl.ds(0, half), col(opp)],
                 wg_ref.at[pl.ds(0, half), col(opp)],
                 send_sems.at[2], recv_sems.at[2], left)
    r_l2.wait_recv()
    r_r2 = _rdma(wg_ref.at[pl.ds(half, half), col(opp)],
                 wg_ref.at[pl.ds(half, half), col(opp)],
                 send_sems.at[3], recv_sems.at[3], right)
    r_r2.wait_recv()

    s_r1.wait_send()
    s_l1.wait_send()
    s_r2.wait_send()
    s_l2.wait_send()
    cp.wait()


def _ag_w(w_shard):
    k, n_per = w_shard.shape
    return pl.pallas_call(
        _ag_w_body,
        out_shape=jax.ShapeDtypeStruct((k, N_DEV * n_per), w_shard.dtype),
        in_specs=[pl.BlockSpec(memory_space=pl.ANY)],
        out_specs=pl.BlockSpec(memory_space=pl.ANY),
        scratch_shapes=[
            pltpu.SemaphoreType.DMA,
            pltpu.SemaphoreType.DMA((4,)),
            pltpu.SemaphoreType.DMA((4,)),
        ],
        compiler_params=pltpu.CompilerParams(collective_id=0),
    )(w_shard)


def _a2a_body(y_ref, out_ref, transit_ref, local_sem, send_sems, recv_sems):
    my = lax.axis_index("i")
    left = (my - 1) % N_DEV
    right = (my + 1) % N_DEV
    opp = (my + 2) % N_DEV
    m_per, n_full = y_ref.shape
    n_per = n_full // N_DEV

    def col(c):
        return pl.ds(c * n_per, n_per)

    def rows(r):
        return pl.ds(r * m_per, m_per)

    _neighbor_barrier(left, right)

    cp = pltpu.make_async_copy(
        y_ref.at[:, col(my)], out_ref.at[rows(my), :], local_sem
    )
    cp.start()

    s_rn = _rdma(y_ref.at[:, col(right)], out_ref.at[rows(my), :],
                 send_sems.at[0], recv_sems.at[0], right)
    s_ln = _rdma(y_ref.at[:, col(left)], out_ref.at[rows(my), :],
                 send_sems.at[1], recv_sems.at[1], left)
    s_rt = _rdma(y_ref.at[:, col(opp)], transit_ref,
                 send_sems.at[2], recv_sems.at[2], right)
    s_rn.start()
    s_ln.start()
    s_rt.start()

    r_t = _rdma(transit_ref, transit_ref,
                send_sems.at[2], recv_sems.at[2], left)
    r_t.wait_recv()
    s_rf = _rdma(transit_ref, out_ref.at[rows(left), :],
                 send_sems.at[3], recv_sems.at[3], right)
    s_rf.start()

    r_ln = _rdma(out_ref.at[rows(left), :], out_ref.at[rows(left), :],
                 send_sems.at[0], recv_sems.at[0], left)
    r_ln.wait_recv()
    r_rn = _rdma(out_ref.at[rows(right), :], out_ref.at[rows(right), :],
                 send_sems.at[1], recv_sems.at[1], right)
    r_rn.wait_recv()
    r_f = _rdma(out_ref.at[rows(opp), :], out_ref.at[rows(opp), :],
                send_sems.at[3], recv_sems.at[3], left)
    r_f.wait_recv()

    s_rn.wait_send()
    s_ln.wait_send()
    s_rt.wait_send()
    s_rf.wait_send()
    cp.wait()


def _a2a(y):
    m_per, n_full = y.shape
    n_per = n_full // N_DEV
    return pl.pallas_call(
        _a2a_body,
        out_shape=jax.ShapeDtypeStruct((N_DEV * m_per, n_per), y.dtype),
        in_specs=[pl.BlockSpec(memory_space=pl.ANY)],
        out_specs=pl.BlockSpec(memory_space=pl.ANY),
        scratch_shapes=[
            pltpu.VMEM((m_per, n_per), y.dtype),
            pltpu.SemaphoreType.DMA,
            pltpu.SemaphoreType.DMA((4,)),
            pltpu.SemaphoreType.DMA((4,)),
        ],
        compiler_params=pltpu.CompilerParams(collective_id=1),
    )(y)


def _gelu(y):
    c = 0.7978845608028654
    return 0.5 * y * (1.0 + jnp.tanh(c * (y + 0.044715 * y * y * y)))


def kernel(x, w_mat):
    w_full = _ag_w(w_mat)
    y = jnp.dot(x, w_full, preferred_element_type=jnp.float32)
    y = _gelu(y).astype(jnp.float32)
    return _a2a(y)
